# Initial kernel scaffold; baseline (speedup 1.0000x reference)
import jax
import jax.numpy as jnp
from jax.experimental import pallas as pl


def kernel(x, gate_W1, gate_b1, gate_W2, gate_b2, W1, b1, W2, b2, W3, b3):
    raise NotImplementedError("write your pallas kernel here")


# speedup vs baseline: 1.5852x; 1.5852x over previous
import jax
import jax.numpy as jnp
from jax.experimental import pallas as pl

_N, _D, _H, _GH, _E = 2048, 768, 128, 64, 16
_BALANCE_COEF = 0.01


def _gate_body(x_ref, w1_ref, b1_ref, w2_ref, b2_ref,
               combine_ref, usage_ref, loss_ref):
    x = x_ref[...]
    gh = jnp.maximum(
        jnp.dot(x, w1_ref[...], preferred_element_type=jnp.float32)
        + b1_ref[...], 0.0)
    logits = (jnp.dot(gh, w2_ref[...], preferred_element_type=jnp.float32)
              + b2_ref[...])
    eid = jax.lax.broadcasted_iota(jnp.int32, logits.shape, 1)
    l1 = jnp.max(logits, axis=1, keepdims=True)
    i1 = jnp.min(jnp.where(logits == l1, eid, _E), axis=1, keepdims=True)
    m1 = eid == i1
    masked = jnp.where(m1, jnp.float32(-1e30), logits)
    l2 = jnp.max(masked, axis=1, keepdims=True)
    i2 = jnp.min(jnp.where(masked == l2, eid, _E), axis=1, keepdims=True)
    m2 = eid == i2
    w1 = 1.0 / (1.0 + jnp.exp(l2 - l1))
    combine_ref[...] = jnp.where(m1, w1, 0.0) + jnp.where(m2, 1.0 - w1, 0.0)
    usage = jnp.sum((m1 | m2).astype(jnp.float32), axis=0,
                    keepdims=True) * (1.0 / _N)
    usage_ref[...] = usage
    loss_ref[...] = (jnp.mean((usage - 1.0 / _E) ** 2)
                     * _BALANCE_COEF).reshape(1, 1)


def _expert_body(x_ref, w1_ref, b1_ref, w2_ref, b2_ref, w3_ref, b3_ref,
                 c_ref, out_ref):
    e = pl.program_id(0)

    @pl.when(e == 0)
    def _():
        out_ref[...] = jnp.zeros_like(out_ref)

    x = x_ref[...]
    h1 = jnp.maximum(
        jnp.dot(x, w1_ref[0], preferred_element_type=jnp.float32)
        + b1_ref[0], 0.0)
    h2 = jnp.maximum(
        jnp.dot(h1, w2_ref[0], preferred_element_type=jnp.float32)
        + b2_ref[0], 0.0)
    call = c_ref[...]
    eid = jax.lax.broadcasted_iota(jnp.int32, call.shape, 1)
    c = jnp.sum(jnp.where(eid == e, call, 0.0), axis=1, keepdims=True)
    y = jnp.dot(h2 * c, w3_ref[0], preferred_element_type=jnp.float32)
    out_ref[...] += y + c * b3_ref[0]


def kernel(x, gate_W1, gate_b1, gate_W2, gate_b2, W1, b1, W2, b2, W3, b3):
    combine, usage, loss = pl.pallas_call(
        _gate_body,
        out_shape=(
            jax.ShapeDtypeStruct((_N, _E), jnp.float32),
            jax.ShapeDtypeStruct((1, _E), jnp.float32),
            jax.ShapeDtypeStruct((1, 1), jnp.float32),
        ),
    )(x, gate_W1, gate_b1.reshape(1, _GH), gate_W2, gate_b2.reshape(1, _E))

    out = pl.pallas_call(
        _expert_body,
        grid=(_E,),
        in_specs=[
            pl.BlockSpec((_N, _D), lambda e: (0, 0)),
            pl.BlockSpec((1, _D, _H), lambda e: (e, 0, 0)),
            pl.BlockSpec((1, 1, _H), lambda e: (e, 0, 0)),
            pl.BlockSpec((1, _H, _H), lambda e: (e, 0, 0)),
            pl.BlockSpec((1, 1, _H), lambda e: (e, 0, 0)),
            pl.BlockSpec((1, _H, _D), lambda e: (e, 0, 0)),
            pl.BlockSpec((1, 1, _D), lambda e: (e, 0, 0)),
            pl.BlockSpec((_N, _E), lambda e: (0, 0)),
        ],
        out_specs=pl.BlockSpec((_N, _D), lambda e: (0, 0)),
        out_shape=jax.ShapeDtypeStruct((_N, _D), jnp.float32),
    )(x, W1, b1.reshape(_E, 1, _H), W2, b2.reshape(_E, 1, _H),
      W3, b3.reshape(_E, 1, _D), combine)

    return out, loss[0, 0], usage.reshape(_E)
